# Initial kernel scaffold; baseline (speedup 1.0000x reference)
import jax
import jax.numpy as jnp
from jax.experimental import pallas as pl


def kernel(node_feature, node_type, edge_time, edge_index, edge_type, W_adapt, b_adapt, Wk, bk, Wq, bq, Wv, bv, Wa, ba, rel_pri, rel_att, rel_msg, skip, ln_g, ln_b, rte_W, rte_b):
    raise NotImplementedError("write your pallas kernel here")


# speedup vs baseline: 3.1778x; 3.1778x over previous
import math
import functools

import jax
import jax.numpy as jnp
import numpy as np
from jax import lax
from jax.experimental import pallas as pl
from jax.experimental.pallas import tpu as pltpu
from jax.experimental.pallas import tpu_sc as plsc

N = 10000
E = 160000
N_HID = 128
NTYPES = 4
NREL = 8
NHEADS = 8
DK = 16
NLAYERS = 2
MAX_T = 240
SQRT_DK = math.sqrt(DK)

NP = 10016
NODE_BLK = 2504
NWORK = 32
CHUNK = 32
NCHUNK = 158
EPW = CHUNK * NCHUNK
EP = EPW * NWORK
ACC_ROWS = 10240
DEN_ROWS = 640


def _rte_emb():
    position = np.arange(MAX_T, dtype=np.float32)[:, None]
    div_term = np.exp(np.arange(0, N_HID, 2, dtype=np.float32) * -(math.log(10000.0) / N_HID))
    emb = np.zeros((MAX_T, N_HID), dtype=np.float32)
    emb[:, 0::2] = np.sin(position * div_term) / math.sqrt(N_HID)
    emb[:, 1::2] = np.cos(position * div_term) / math.sqrt(N_HID)
    return emb


_EMB = _rte_emb()

_SEL8 = np.zeros((NHEADS, N_HID), np.float32)
for _h in range(NHEADS):
    _SEL8[_h, _h * DK:(_h + 1) * DK] = 1.0


def _erf(z):
    az = jnp.abs(z)
    t = 1.0 / (1.0 + 0.3275911 * az)
    poly = t * (0.254829592 + t * (-0.284496736 + t * (1.421413741
               + t * (-1.453152027 + t * 1.061405429))))
    y = 1.0 - poly * jnp.exp(-az * az)
    return jnp.sign(z) * y


def _gelu(x):
    return 0.5 * x * (1.0 + _erf(x * (1.0 / math.sqrt(2.0))))


def _adapt_body(nf_ref, nt_ref, w_ref, b_ref, out_ref):
    x = nf_ref[...]
    nt = nt_ref[...]
    acc = jnp.zeros((NODE_BLK, N_HID), jnp.float32)
    for t in range(NTYPES):
        y = jnp.tanh(jnp.dot(x, w_ref[t], preferred_element_type=jnp.float32) + b_ref[t])
        acc = jnp.where(nt == t, y, acc)
    out_ref[...] = acc


def _adapt(nf, nt, W, b):
    grid = NP // NODE_BLK
    return pl.pallas_call(
        _adapt_body,
        grid=(grid,),
        in_specs=[
            pl.BlockSpec((NODE_BLK, N_HID), lambda i: (i, 0)),
            pl.BlockSpec((NODE_BLK, 1), lambda i: (i, 0)),
            pl.BlockSpec((NTYPES, N_HID, N_HID), lambda i: (0, 0, 0)),
            pl.BlockSpec((NTYPES, N_HID), lambda i: (0, 0)),
        ],
        out_specs=pl.BlockSpec((NODE_BLK, N_HID), lambda i: (i, 0)),
        out_shape=jax.ShapeDtypeStruct((NP, N_HID), jnp.float32),
    )(nf, nt, W, b)


def _proj_q_body(x_ref, nt_ref, wq_ref, bq_ref, ba_ref, qa_ref):
    x = x_ref[...]
    nt = nt_ref[...]
    q = jnp.zeros((NODE_BLK, N_HID), jnp.float32)
    for t in range(NTYPES):
        q = jnp.where(nt == t,
                      jnp.dot(x, wq_ref[t], preferred_element_type=jnp.float32) + bq_ref[t], q)
    qa_ref[...] = jnp.stack(
        [jnp.dot(q, ba_ref[r], preferred_element_type=jnp.float32) for r in range(NREL)],
        axis=1)


def _proj_q(x, nt, Wq, bq, BA):
    grid = NP // NODE_BLK
    full3 = lambda i: (0, 0, 0)
    return pl.pallas_call(
        _proj_q_body,
        grid=(grid,),
        in_specs=[
            pl.BlockSpec((NODE_BLK, N_HID), lambda i: (i, 0)),
            pl.BlockSpec((NODE_BLK, 1), lambda i: (i, 0)),
            pl.BlockSpec((NTYPES, N_HID, N_HID), full3),
            pl.BlockSpec((NTYPES, N_HID), lambda i: (0, 0)),
            pl.BlockSpec((NREL, N_HID, N_HID), full3),
        ],
        out_specs=pl.BlockSpec((NODE_BLK, NREL, N_HID), lambda i: (i, 0, 0)),
        out_shape=jax.ShapeDtypeStruct((NP, NREL, N_HID), jnp.float32),
    )(x, nt, Wq, bq, BA)


def _proj_c_body(x_ref, nt_ref, wk_ref, wv_ref, bm_ref, c_ref):
    x = x_ref[...]
    nt = nt_ref[...]
    kx = jnp.zeros((NODE_BLK, N_HID), jnp.float32)
    vx = jnp.zeros((NODE_BLK, N_HID), jnp.float32)
    for t in range(NTYPES):
        m = nt == t
        kx = jnp.where(m, jnp.dot(x, wk_ref[t], preferred_element_type=jnp.float32), kx)
        vx = jnp.where(m, jnp.dot(x, wv_ref[t], preferred_element_type=jnp.float32), vx)
    kbc = jnp.broadcast_to(kx[:, None, :], (NODE_BLK, NREL, N_HID))
    vr = jnp.stack(
        [jnp.dot(vx, bm_ref[r], preferred_element_type=jnp.float32) for r in range(NREL)],
        axis=1)
    c_ref[...] = jnp.concatenate([kbc, vr], axis=-1)


def _proj_c(x, nt, Wk, Wv, BM):
    grid = NP // NODE_BLK
    full3 = lambda i: (0, 0, 0)
    return pl.pallas_call(
        _proj_c_body,
        grid=(grid,),
        in_specs=[
            pl.BlockSpec((NODE_BLK, N_HID), lambda i: (i, 0)),
            pl.BlockSpec((NODE_BLK, 1), lambda i: (i, 0)),
            pl.BlockSpec((NTYPES, N_HID, N_HID), full3),
            pl.BlockSpec((NTYPES, N_HID, N_HID), full3),
            pl.BlockSpec((NREL, N_HID, N_HID), full3),
        ],
        out_specs=pl.BlockSpec((NODE_BLK, NREL, 2 * N_HID), lambda i: (i, 0, 0)),
        out_shape=jax.ShapeDtypeStruct((NP, NREL, 2 * N_HID), jnp.float32),
        compiler_params=pltpu.CompilerParams(vmem_limit_bytes=64 * 1024 * 1024),
    )(x, nt, Wk, Wv, BM)


def _tables_body(emb_ref, rw_ref, rb_ref, wk_ref, bk_ref, wv_ref, bv_ref,
                 bm_ref, out_ref):
    temb = jnp.dot(emb_ref[...], rw_ref[...], preferred_element_type=jnp.float32) + rb_ref[0]
    for st in range(NTYPES):
        tk = jnp.dot(temb, wk_ref[st], preferred_element_type=jnp.float32) + bk_ref[st]
        tv0 = jnp.dot(temb, wv_ref[st], preferred_element_type=jnp.float32) + bv_ref[st]
        for r in range(NREL):
            tvr = jnp.dot(tv0, bm_ref[r], preferred_element_type=jnp.float32)
            out_ref[st * NREL + r] = jnp.concatenate([tk, tvr], axis=-1)


def _tables(rte_W, rte_b, Wk, bk, Wv, bv, BM):
    full3 = lambda: (0, 0, 0)
    return pl.pallas_call(
        _tables_body,
        in_specs=[
            pl.BlockSpec((MAX_T, N_HID), lambda: (0, 0)),
            pl.BlockSpec((N_HID, N_HID), lambda: (0, 0)),
            pl.BlockSpec((1, N_HID), lambda: (0, 0)),
            pl.BlockSpec((NTYPES, N_HID, N_HID), full3),
            pl.BlockSpec((NTYPES, N_HID), lambda: (0, 0)),
            pl.BlockSpec((NTYPES, N_HID, N_HID), full3),
            pl.BlockSpec((NTYPES, N_HID), lambda: (0, 0)),
            pl.BlockSpec((NREL, N_HID, N_HID), full3),
        ],
        out_specs=pl.BlockSpec((NTYPES * NREL, MAX_T, 2 * N_HID), full3),
        out_shape=jax.ShapeDtypeStruct((NTYPES * NREL, MAX_T, 2 * N_HID), jnp.float32),
    )(_EMB, rte_W, rte_b.reshape(1, N_HID), Wk, bk, Wv, bv, BM)


def _edge_body(qa_hbm, c_hbm, t_hbm, src_hbm, dst_hbm, time_hbm, rt_hbm, nt_hbm,
               z_hbm, out_hbm, den_hbm,
               nt_v, src_v, dst_v, time_v, rt_v, qi_v, ci_v, ti_v, drow_v,
               buf_q, buf_c, buf_t, out_buf, wbuf, acc_ref, den_ref, sem):
    core = lax.axis_index("c")
    sub = lax.axis_index("s")
    wid = sub * 2 + core
    zero16 = jnp.zeros((16,), jnp.float32)

    pltpu.sync_copy(z_hbm, out_buf)
    pltpu.sync_copy(z_hbm, wbuf)
    pltpu.sync_copy(nt_hbm, nt_v)
    rows_per_tile = ACC_ROWS // 16
    for j in range(rows_per_tile // CHUNK):
        pltpu.sync_copy(out_buf, acc_ref.at[pl.ds(sub * rows_per_tile + j * CHUNK, CHUNK)])
    for j in range(DEN_ROWS // CHUNK):
        @pl.when(sub == j % 16)
        def _():
            pltpu.sync_copy(out_buf, den_ref.at[pl.ds(j * CHUNK, CHUNK)])
    plsc.subcore_barrier()

    lanes = lax.iota(jnp.int32, 16)
    lmask = lanes < NHEADS

    def chunk(cidx, _):
        base = wid * EPW + cidx * CHUNK
        pltpu.sync_copy(src_hbm.at[pl.ds(base, CHUNK)], src_v)
        pltpu.sync_copy(dst_hbm.at[pl.ds(base, CHUNK)], dst_v)
        pltpu.sync_copy(time_hbm.at[pl.ds(base, CHUNK)], time_v)
        pltpu.sync_copy(rt_hbm.at[pl.ds(base, CHUNK)], rt_v)
        for j in range(CHUNK // 16):
            sl = pl.ds(j * 16, 16)
            s16 = src_v[sl]
            d16 = dst_v[sl]
            t16 = time_v[sl]
            r16 = rt_v[sl]
            st16 = plsc.load_gather(nt_v, [s16])
            qi_v[sl] = d16 * NREL + r16
            ci_v[sl] = s16 * NREL + r16
            ti_v[sl] = (st16 * NREL + r16) * MAX_T + t16
            drow_v[sl] = d16 // 16
        cq = pltpu.async_copy(qa_hbm.at[qi_v], buf_q, sem)
        cc = pltpu.async_copy(c_hbm.at[ci_v], buf_c, sem)
        ct = pltpu.async_copy(t_hbm.at[ti_v], buf_t, sem)
        cq.wait()
        cc.wait()
        ct.wait()

        for g in range(CHUNK // 16):
            d16 = dst_v[pl.ds(g * 16, 16)]
            for i in range(16):
                e = g * 16 + i
                wacc = zero16
                for h in range(NHEADS):
                    hs = pl.ds(h * DK, 16)
                    vs = pl.ds(N_HID + h * DK, 16)
                    q = buf_q[e, hs]
                    k = buf_c[e, hs] + buf_t[e, hs]
                    s = jnp.sum(q * k)
                    w = jnp.exp(jnp.full((16,), s, jnp.float32))
                    v = buf_c[e, vs] + buf_t[e, vs]
                    out_buf[e, hs] = v * w
                    wacc = jnp.where(lanes == h, w, wacc)
                erow = jnp.full((16,), e, jnp.int32)
                ecol = jnp.full((16,), (d16[i] % 16) * NHEADS, jnp.int32) + lanes
                plsc.store_scatter(wbuf, [erow, ecol], wacc, mask=lmask)
        pltpu.sync_copy(out_buf, acc_ref.at[dst_v], add=True)
        pltpu.sync_copy(wbuf, den_ref.at[drow_v], add=True)
        pltpu.sync_copy(z_hbm, wbuf)
        return 0

    lax.fori_loop(0, NCHUNK, chunk, 0)
    plsc.subcore_barrier()

    for j in range(rows_per_tile // CHUNK):
        row = sub * rows_per_tile + j * CHUNK
        pltpu.sync_copy(acc_ref.at[pl.ds(row, CHUNK)], out_buf)
        pltpu.sync_copy(out_buf, out_hbm.at[core].at[pl.ds(row, CHUNK)])
    for j in range(DEN_ROWS // CHUNK):
        @pl.when(sub == j % 16)
        def _():
            pltpu.sync_copy(den_ref.at[pl.ds(j * CHUNK, CHUNK)], wbuf)
            pltpu.sync_copy(wbuf, den_hbm.at[core].at[pl.ds(j * CHUNK, CHUNK)])


def _edge_sc(qa, c, t, src, dst, time, rt, nt, z):
    mesh = plsc.VectorSubcoreMesh(core_axis_name="c", subcore_axis_name="s")
    f = pl.kernel(
        _edge_body,
        out_type=[
            pltpu.HBM((2, ACC_ROWS, N_HID), jnp.float32),
            pltpu.HBM((2, DEN_ROWS, N_HID), jnp.float32),
        ],
        mesh=mesh,
        scratch_types=[
            pltpu.VMEM((NP,), jnp.int32),
            pltpu.VMEM((CHUNK,), jnp.int32),
            pltpu.VMEM((CHUNK,), jnp.int32),
            pltpu.VMEM((CHUNK,), jnp.int32),
            pltpu.VMEM((CHUNK,), jnp.int32),
            pltpu.VMEM((CHUNK,), jnp.int32),
            pltpu.VMEM((CHUNK,), jnp.int32),
            pltpu.VMEM((CHUNK,), jnp.int32),
            pltpu.VMEM((CHUNK,), jnp.int32),
            pltpu.VMEM((CHUNK, N_HID), jnp.float32),
            pltpu.VMEM((CHUNK, 2 * N_HID), jnp.float32),
            pltpu.VMEM((CHUNK, 2 * N_HID), jnp.float32),
            pltpu.VMEM((CHUNK, N_HID), jnp.float32),
            pltpu.VMEM((CHUNK, N_HID), jnp.float32),
            pltpu.VMEM_SHARED((ACC_ROWS, N_HID), jnp.float32),
            pltpu.VMEM_SHARED((DEN_ROWS, N_HID), jnp.float32),
            pltpu.SemaphoreType.DMA,
        ],
        compiler_params=pltpu.CompilerParams(needs_layout_passes=False),
    )
    return f(qa, c, t, src, dst, time, rt, nt, z)


def _update_body(a_ref, d_ref, x_ref, nt_ref, sel_ref, wa_ref, ba_ref,
                 beta_ref, lg_ref, lb_ref, out_ref):
    num = a_ref[0] + a_ref[1]
    d8 = jnp.sum(d_ref[...], axis=0)
    den = jnp.dot(d8, sel_ref[...], preferred_element_type=jnp.float32) + 1e-16
    g = _gelu(num / den)
    x = x_ref[...]
    nt = nt_ref[...]
    acc = jnp.zeros((NODE_BLK, N_HID), jnp.float32)
    for t in range(NTYPES):
        out_t = (jnp.dot(g, wa_ref[t], preferred_element_type=jnp.float32)
                 + ba_ref[t] + x * beta_ref[t])
        mu = jnp.mean(out_t, axis=-1, keepdims=True)
        var = jnp.mean((out_t - mu) * (out_t - mu), axis=-1, keepdims=True)
        y = (out_t - mu) * lax.rsqrt(var + 1e-5) * lg_ref[t] + lb_ref[t]
        acc = jnp.where(nt == t, y, acc)
    out_ref[...] = acc


def _update(agg2, den, x, nt, Wa_eff, ba_eff, beta, lg, lb):
    grid = NP // NODE_BLK
    return pl.pallas_call(
        _update_body,
        grid=(grid,),
        in_specs=[
            pl.BlockSpec((2, NODE_BLK, N_HID), lambda i: (0, i, 0)),
            pl.BlockSpec((2, NODE_BLK, NHEADS), lambda i: (0, i, 0)),
            pl.BlockSpec((NODE_BLK, N_HID), lambda i: (i, 0)),
            pl.BlockSpec((NODE_BLK, 1), lambda i: (i, 0)),
            pl.BlockSpec((NHEADS, N_HID), lambda i: (0, 0)),
            pl.BlockSpec((NTYPES, N_HID, N_HID), lambda i: (0, 0, 0)),
            pl.BlockSpec((NTYPES, N_HID), lambda i: (0, 0)),
            pl.BlockSpec((NTYPES, N_HID), lambda i: (0, 0)),
            pl.BlockSpec((NTYPES, N_HID), lambda i: (0, 0)),
            pl.BlockSpec((NTYPES, N_HID), lambda i: (0, 0)),
        ],
        out_specs=pl.BlockSpec((NODE_BLK, N_HID), lambda i: (i, 0)),
        out_shape=jax.ShapeDtypeStruct((NP, N_HID), jnp.float32),
    )(agg2, den, x, nt, _SEL8, Wa_eff, ba_eff, beta, lg, lb)


def _block_diag(mats, scale=None):
    out = jnp.zeros((NREL, N_HID, N_HID), jnp.float32)
    for h in range(NHEADS):
        blk = mats[:, h]
        if scale is not None:
            blk = jnp.transpose(blk, (0, 2, 1)) * scale[:, h][:, None, None]
        out = out.at[:, h * DK:(h + 1) * DK, h * DK:(h + 1) * DK].set(blk)
    return out


def kernel(node_feature, node_type, edge_time, edge_index, edge_type,
           W_adapt, b_adapt, Wk, bk, Wq, bq, Wv, bv, Wa, ba, rel_pri,
           rel_att, rel_msg, skip, ln_g, ln_b, rte_W, rte_b):
    nt = node_type.astype(jnp.int32)
    nf = jnp.concatenate([node_feature, jnp.zeros((NP - N, N_HID), jnp.float32)])
    nt_p = jnp.concatenate([nt, jnp.zeros((NP - N,), jnp.int32)])
    nt_col = nt_p.reshape(NP, 1)

    pad = EP - E
    src = jnp.concatenate([edge_index[0].astype(jnp.int32), jnp.zeros((pad,), jnp.int32)])
    dst = jnp.concatenate([edge_index[1].astype(jnp.int32), jnp.full((pad,), N, jnp.int32)])
    tim = jnp.concatenate([edge_time.astype(jnp.int32), jnp.zeros((pad,), jnp.int32)])
    rt = jnp.concatenate([edge_type.astype(jnp.int32), jnp.zeros((pad,), jnp.int32)])

    zpage = jnp.zeros((CHUNK, N_HID), jnp.float32)

    x = _adapt(nf, nt_col, W_adapt, b_adapt)

    for l in range(NLAYERS):
        BA = _block_diag(rel_att[l], scale=rel_pri[l] / SQRT_DK)
        BM = _block_diag(rel_msg[l])
        qa = _proj_q(x, nt_col, Wq[l], bq[l], BA)
        c = _proj_c(x, nt_col, Wk[l], Wv[l], BM)
        tbl = _tables(rte_W[l], rte_b[l], Wk[l], bk[l], Wv[l], bv[l], BM)
        agg2, den = _edge_sc(
            qa.reshape(NP * NREL, N_HID),
            c.reshape(NP * NREL, 2 * N_HID),
            tbl.reshape(NTYPES * NREL * MAX_T, 2 * N_HID),
            src, dst, tim, rt, nt_p, zpage)
        alpha = jax.nn.sigmoid(skip[l])
        Wa_eff = Wa[l] * alpha[:, None, None]
        ba_eff = ba[l] * alpha[:, None]
        beta = jnp.broadcast_to((1.0 - alpha)[:, None], (NTYPES, N_HID))
        den3 = den.reshape(2, DEN_ROWS * N_HID)[:, :NP * NHEADS]
        x = _update(agg2, den3.reshape(2, NP, NHEADS), x, nt_col,
                    Wa_eff, ba_eff, beta, ln_g[l], ln_b[l])

    return x[:N]


# speedup vs baseline: 3.7492x; 1.1798x over previous
import math
import functools

import jax
import jax.numpy as jnp
import numpy as np
from jax import lax
from jax.experimental import pallas as pl
from jax.experimental.pallas import tpu as pltpu
from jax.experimental.pallas import tpu_sc as plsc

N = 10000
E = 160000
N_HID = 128
NTYPES = 4
NREL = 8
NHEADS = 8
DK = 16
NLAYERS = 2
MAX_T = 240
SQRT_DK = math.sqrt(DK)

NP = 10016
NODE_BLK = 2504
NWORK = 32
CHUNK = 32
NCHUNK = 158
EPW = CHUNK * NCHUNK
EP = EPW * NWORK
ACC_ROWS = 10240
DEN_ROWS = 640


def _rte_emb():
    position = np.arange(MAX_T, dtype=np.float32)[:, None]
    div_term = np.exp(np.arange(0, N_HID, 2, dtype=np.float32) * -(math.log(10000.0) / N_HID))
    emb = np.zeros((MAX_T, N_HID), dtype=np.float32)
    emb[:, 0::2] = np.sin(position * div_term) / math.sqrt(N_HID)
    emb[:, 1::2] = np.cos(position * div_term) / math.sqrt(N_HID)
    return emb


_EMB = _rte_emb()

_SEL8 = np.zeros((NHEADS, N_HID), np.float32)
for _h in range(NHEADS):
    _SEL8[_h, _h * DK:(_h + 1) * DK] = 1.0


def _erf(z):
    az = jnp.abs(z)
    t = 1.0 / (1.0 + 0.3275911 * az)
    poly = t * (0.254829592 + t * (-0.284496736 + t * (1.421413741
               + t * (-1.453152027 + t * 1.061405429))))
    y = 1.0 - poly * jnp.exp(-az * az)
    return jnp.sign(z) * y


def _gelu(x):
    return 0.5 * x * (1.0 + _erf(x * (1.0 / math.sqrt(2.0))))


def _adapt_body(nf_ref, nt_ref, w_ref, b_ref, out_ref):
    x = nf_ref[...]
    nt = nt_ref[...]
    acc = jnp.zeros((NODE_BLK, N_HID), jnp.float32)
    for t in range(NTYPES):
        y = jnp.tanh(jnp.dot(x, w_ref[t], preferred_element_type=jnp.float32) + b_ref[t])
        acc = jnp.where(nt == t, y, acc)
    out_ref[...] = acc


def _adapt(nf, nt, W, b):
    grid = NP // NODE_BLK
    return pl.pallas_call(
        _adapt_body,
        grid=(grid,),
        in_specs=[
            pl.BlockSpec((NODE_BLK, N_HID), lambda i: (i, 0)),
            pl.BlockSpec((NODE_BLK, 1), lambda i: (i, 0)),
            pl.BlockSpec((NTYPES, N_HID, N_HID), lambda i: (0, 0, 0)),
            pl.BlockSpec((NTYPES, N_HID), lambda i: (0, 0)),
        ],
        out_specs=pl.BlockSpec((NODE_BLK, N_HID), lambda i: (i, 0)),
        out_shape=jax.ShapeDtypeStruct((NP, N_HID), jnp.float32),
    )(nf, nt, W, b)


def _proj_q_body(x_ref, nt_ref, wq_ref, bq_ref, ba_ref, qa_ref):
    x = x_ref[...]
    nt = nt_ref[...]
    q = jnp.zeros((NODE_BLK, N_HID), jnp.float32)
    for t in range(NTYPES):
        q = jnp.where(nt == t,
                      jnp.dot(x, wq_ref[t], preferred_element_type=jnp.float32) + bq_ref[t], q)
    qa_ref[...] = jnp.stack(
        [jnp.dot(q, ba_ref[r], preferred_element_type=jnp.float32) for r in range(NREL)],
        axis=1)


def _proj_q(x, nt, Wq, bq, BA):
    grid = NP // NODE_BLK
    full3 = lambda i: (0, 0, 0)
    return pl.pallas_call(
        _proj_q_body,
        grid=(grid,),
        in_specs=[
            pl.BlockSpec((NODE_BLK, N_HID), lambda i: (i, 0)),
            pl.BlockSpec((NODE_BLK, 1), lambda i: (i, 0)),
            pl.BlockSpec((NTYPES, N_HID, N_HID), full3),
            pl.BlockSpec((NTYPES, N_HID), lambda i: (0, 0)),
            pl.BlockSpec((NREL, N_HID, N_HID), full3),
        ],
        out_specs=pl.BlockSpec((NODE_BLK, NREL, N_HID), lambda i: (i, 0, 0)),
        out_shape=jax.ShapeDtypeStruct((NP, NREL, N_HID), jnp.float32),
    )(x, nt, Wq, bq, BA)


def _proj_c_body(x_ref, nt_ref, wk_ref, wv_ref, bm_ref, c_ref):
    x = x_ref[...]
    nt = nt_ref[...]
    kx = jnp.zeros((NODE_BLK, N_HID), jnp.float32)
    vx = jnp.zeros((NODE_BLK, N_HID), jnp.float32)
    for t in range(NTYPES):
        m = nt == t
        kx = jnp.where(m, jnp.dot(x, wk_ref[t], preferred_element_type=jnp.float32), kx)
        vx = jnp.where(m, jnp.dot(x, wv_ref[t], preferred_element_type=jnp.float32), vx)
    kbc = jnp.broadcast_to(kx[:, None, :], (NODE_BLK, NREL, N_HID))
    vr = jnp.stack(
        [jnp.dot(vx, bm_ref[r], preferred_element_type=jnp.float32) for r in range(NREL)],
        axis=1)
    c_ref[...] = jnp.concatenate([kbc, vr], axis=-1)


def _proj_c(x, nt, Wk, Wv, BM):
    grid = NP // NODE_BLK
    full3 = lambda i: (0, 0, 0)
    return pl.pallas_call(
        _proj_c_body,
        grid=(grid,),
        in_specs=[
            pl.BlockSpec((NODE_BLK, N_HID), lambda i: (i, 0)),
            pl.BlockSpec((NODE_BLK, 1), lambda i: (i, 0)),
            pl.BlockSpec((NTYPES, N_HID, N_HID), full3),
            pl.BlockSpec((NTYPES, N_HID, N_HID), full3),
            pl.BlockSpec((NREL, N_HID, N_HID), full3),
        ],
        out_specs=pl.BlockSpec((NODE_BLK, NREL, 2 * N_HID), lambda i: (i, 0, 0)),
        out_shape=jax.ShapeDtypeStruct((NP, NREL, 2 * N_HID), jnp.float32),
        compiler_params=pltpu.CompilerParams(vmem_limit_bytes=64 * 1024 * 1024),
    )(x, nt, Wk, Wv, BM)


def _tables_body(emb_ref, rw_ref, rb_ref, wk_ref, bk_ref, wv_ref, bv_ref,
                 bm_ref, out_ref):
    temb = jnp.dot(emb_ref[...], rw_ref[...], preferred_element_type=jnp.float32) + rb_ref[0]
    for st in range(NTYPES):
        tk = jnp.dot(temb, wk_ref[st], preferred_element_type=jnp.float32) + bk_ref[st]
        tv0 = jnp.dot(temb, wv_ref[st], preferred_element_type=jnp.float32) + bv_ref[st]
        for r in range(NREL):
            tvr = jnp.dot(tv0, bm_ref[r], preferred_element_type=jnp.float32)
            out_ref[st * NREL + r] = jnp.concatenate([tk, tvr], axis=-1)


def _tables(rte_W, rte_b, Wk, bk, Wv, bv, BM):
    full3 = lambda: (0, 0, 0)
    return pl.pallas_call(
        _tables_body,
        in_specs=[
            pl.BlockSpec((MAX_T, N_HID), lambda: (0, 0)),
            pl.BlockSpec((N_HID, N_HID), lambda: (0, 0)),
            pl.BlockSpec((1, N_HID), lambda: (0, 0)),
            pl.BlockSpec((NTYPES, N_HID, N_HID), full3),
            pl.BlockSpec((NTYPES, N_HID), lambda: (0, 0)),
            pl.BlockSpec((NTYPES, N_HID, N_HID), full3),
            pl.BlockSpec((NTYPES, N_HID), lambda: (0, 0)),
            pl.BlockSpec((NREL, N_HID, N_HID), full3),
        ],
        out_specs=pl.BlockSpec((NTYPES * NREL, MAX_T, 2 * N_HID), full3),
        out_shape=jax.ShapeDtypeStruct((NTYPES * NREL, MAX_T, 2 * N_HID), jnp.float32),
    )(_EMB, rte_W, rte_b.reshape(1, N_HID), Wk, bk, Wv, bv, BM)


def _edge_body(qa_hbm, c_hbm, t_hbm, src_hbm, dst_hbm, time_hbm, rt_hbm, nt_hbm,
               z_hbm, out_hbm, den_hbm,
               nt_v, src_v, dst_v, time_v, rt_v, qi_v, ci_v, ti_v, drow_v,
               buf_q, buf_c, buf_t, out_buf, wbuf, acc_ref, den_ref,
               sem, semi, sems):
    core = lax.axis_index("c")
    sub = lax.axis_index("s")
    wid = sub * 2 + core
    zero16 = jnp.zeros((16,), jnp.float32)

    pltpu.sync_copy(z_hbm, out_buf)
    pltpu.sync_copy(z_hbm, wbuf)
    pltpu.sync_copy(nt_hbm, nt_v)
    rows_per_tile = ACC_ROWS // 16
    for j in range(rows_per_tile // CHUNK):
        pltpu.sync_copy(out_buf, acc_ref.at[pl.ds(sub * rows_per_tile + j * CHUNK, CHUNK)])
    for j in range(DEN_ROWS // CHUNK):
        @pl.when(sub == j % 16)
        def _():
            pltpu.sync_copy(out_buf, den_ref.at[pl.ds(j * CHUNK, CHUNK)])
    plsc.subcore_barrier()

    lanes = lax.iota(jnp.int32, 16)
    lmask = lanes < NHEADS

    def chunk(cidx, _):
        base = wid * EPW + cidx * CHUNK
        c1 = pltpu.async_copy(src_hbm.at[pl.ds(base, CHUNK)], src_v, semi)
        c2 = pltpu.async_copy(dst_hbm.at[pl.ds(base, CHUNK)], dst_v, semi)
        c3 = pltpu.async_copy(time_hbm.at[pl.ds(base, CHUNK)], time_v, semi)
        c4 = pltpu.async_copy(rt_hbm.at[pl.ds(base, CHUNK)], rt_v, semi)

        @pl.when(cidx > 0)
        def _():
            pltpu.make_async_copy(z_hbm, out_buf, sems).wait()
            pltpu.make_async_copy(z_hbm, wbuf, sems).wait()
        cz = pltpu.async_copy(z_hbm, wbuf, semi)

        c1.wait()
        c2.wait()
        c3.wait()
        c4.wait()
        cz.wait()
        for j in range(CHUNK // 16):
            sl = pl.ds(j * 16, 16)
            s16 = src_v[sl]
            d16 = dst_v[sl]
            t16 = time_v[sl]
            r16 = rt_v[sl]
            st16 = plsc.load_gather(nt_v, [s16])
            qi_v[sl] = d16 * NREL + r16
            ci_v[sl] = s16 * NREL + r16
            ti_v[sl] = (st16 * NREL + r16) * MAX_T + t16
            drow_v[sl] = d16 // 16
        cq = pltpu.async_copy(qa_hbm.at[qi_v], buf_q, sem)
        cc = pltpu.async_copy(c_hbm.at[ci_v], buf_c, sem)
        ct = pltpu.async_copy(t_hbm.at[ti_v], buf_t, sem)
        cq.wait()
        cc.wait()
        ct.wait()

        for g in range(CHUNK // 16):
            d16 = dst_v[pl.ds(g * 16, 16)]
            for i in range(16):
                e = g * 16 + i
                wacc = zero16
                for h in range(NHEADS):
                    hs = pl.ds(h * DK, 16)
                    vs = pl.ds(N_HID + h * DK, 16)
                    q = buf_q[e, hs]
                    k = buf_c[e, hs] + buf_t[e, hs]
                    s = jnp.sum(q * k)
                    w = jnp.exp(jnp.full((16,), s, jnp.float32))
                    v = buf_c[e, vs] + buf_t[e, vs]
                    out_buf[e, hs] = v * w
                    wacc = jnp.where(lanes == h, w, wacc)
                erow = jnp.full((16,), e, jnp.int32)
                ecol = jnp.full((16,), (d16[i] % 16) * NHEADS, jnp.int32) + lanes
                plsc.store_scatter(wbuf, [erow, ecol], wacc, mask=lmask)
        pltpu.async_copy(out_buf, acc_ref.at[dst_v], sems, add=True)
        pltpu.async_copy(wbuf, den_ref.at[drow_v], sems, add=True)
        return 0

    lax.fori_loop(0, NCHUNK, chunk, 0)
    pltpu.make_async_copy(z_hbm, out_buf, sems).wait()
    pltpu.make_async_copy(z_hbm, wbuf, sems).wait()
    plsc.subcore_barrier()

    for j in range(rows_per_tile // CHUNK):
        row = sub * rows_per_tile + j * CHUNK
        pltpu.sync_copy(acc_ref.at[pl.ds(row, CHUNK)], out_buf)
        pltpu.sync_copy(out_buf, out_hbm.at[core].at[pl.ds(row, CHUNK)])
    for j in range(DEN_ROWS // CHUNK):
        @pl.when(sub == j % 16)
        def _():
            pltpu.sync_copy(den_ref.at[pl.ds(j * CHUNK, CHUNK)], wbuf)
            pltpu.sync_copy(wbuf, den_hbm.at[core].at[pl.ds(j * CHUNK, CHUNK)])


def _edge_sc(qa, c, t, src, dst, time, rt, nt, z):
    mesh = plsc.VectorSubcoreMesh(core_axis_name="c", subcore_axis_name="s")
    f = pl.kernel(
        _edge_body,
        out_type=[
            pltpu.HBM((2, ACC_ROWS, N_HID), jnp.float32),
            pltpu.HBM((2, DEN_ROWS, N_HID), jnp.float32),
        ],
        mesh=mesh,
        scratch_types=[
            pltpu.VMEM((NP,), jnp.int32),
            pltpu.VMEM((CHUNK,), jnp.int32),
            pltpu.VMEM((CHUNK,), jnp.int32),
            pltpu.VMEM((CHUNK,), jnp.int32),
            pltpu.VMEM((CHUNK,), jnp.int32),
            pltpu.VMEM((CHUNK,), jnp.int32),
            pltpu.VMEM((CHUNK,), jnp.int32),
            pltpu.VMEM((CHUNK,), jnp.int32),
            pltpu.VMEM((CHUNK,), jnp.int32),
            pltpu.VMEM((CHUNK, N_HID), jnp.float32),
            pltpu.VMEM((CHUNK, 2 * N_HID), jnp.float32),
            pltpu.VMEM((CHUNK, 2 * N_HID), jnp.float32),
            pltpu.VMEM((CHUNK, N_HID), jnp.float32),
            pltpu.VMEM((CHUNK, N_HID), jnp.float32),
            pltpu.VMEM_SHARED((ACC_ROWS, N_HID), jnp.float32),
            pltpu.VMEM_SHARED((DEN_ROWS, N_HID), jnp.float32),
            pltpu.SemaphoreType.DMA,
            pltpu.SemaphoreType.DMA,
            pltpu.SemaphoreType.DMA,
        ],
        compiler_params=pltpu.CompilerParams(needs_layout_passes=False),
    )
    return f(qa, c, t, src, dst, time, rt, nt, z)


def _update_body(a_ref, d_ref, x_ref, nt_ref, sel_ref, wa_ref, ba_ref,
                 beta_ref, lg_ref, lb_ref, out_ref):
    num = a_ref[0] + a_ref[1]
    d8 = jnp.sum(d_ref[...], axis=0)
    den = jnp.dot(d8, sel_ref[...], preferred_element_type=jnp.float32) + 1e-16
    g = _gelu(num / den)
    x = x_ref[...]
    nt = nt_ref[...]
    acc = jnp.zeros((NODE_BLK, N_HID), jnp.float32)
    for t in range(NTYPES):
        out_t = (jnp.dot(g, wa_ref[t], preferred_element_type=jnp.float32)
                 + ba_ref[t] + x * beta_ref[t])
        mu = jnp.mean(out_t, axis=-1, keepdims=True)
        var = jnp.mean((out_t - mu) * (out_t - mu), axis=-1, keepdims=True)
        y = (out_t - mu) * lax.rsqrt(var + 1e-5) * lg_ref[t] + lb_ref[t]
        acc = jnp.where(nt == t, y, acc)
    out_ref[...] = acc


def _update(agg2, den, x, nt, Wa_eff, ba_eff, beta, lg, lb):
    grid = NP // NODE_BLK
    return pl.pallas_call(
        _update_body,
        grid=(grid,),
        in_specs=[
            pl.BlockSpec((2, NODE_BLK, N_HID), lambda i: (0, i, 0)),
            pl.BlockSpec((2, NODE_BLK, NHEADS), lambda i: (0, i, 0)),
            pl.BlockSpec((NODE_BLK, N_HID), lambda i: (i, 0)),
            pl.BlockSpec((NODE_BLK, 1), lambda i: (i, 0)),
            pl.BlockSpec((NHEADS, N_HID), lambda i: (0, 0)),
            pl.BlockSpec((NTYPES, N_HID, N_HID), lambda i: (0, 0, 0)),
            pl.BlockSpec((NTYPES, N_HID), lambda i: (0, 0)),
            pl.BlockSpec((NTYPES, N_HID), lambda i: (0, 0)),
            pl.BlockSpec((NTYPES, N_HID), lambda i: (0, 0)),
            pl.BlockSpec((NTYPES, N_HID), lambda i: (0, 0)),
        ],
        out_specs=pl.BlockSpec((NODE_BLK, N_HID), lambda i: (i, 0)),
        out_shape=jax.ShapeDtypeStruct((NP, N_HID), jnp.float32),
    )(agg2, den, x, nt, _SEL8, Wa_eff, ba_eff, beta, lg, lb)


def _block_diag(mats, scale=None):
    out = jnp.zeros((NREL, N_HID, N_HID), jnp.float32)
    for h in range(NHEADS):
        blk = mats[:, h]
        if scale is not None:
            blk = jnp.transpose(blk, (0, 2, 1)) * scale[:, h][:, None, None]
        out = out.at[:, h * DK:(h + 1) * DK, h * DK:(h + 1) * DK].set(blk)
    return out


def kernel(node_feature, node_type, edge_time, edge_index, edge_type,
           W_adapt, b_adapt, Wk, bk, Wq, bq, Wv, bv, Wa, ba, rel_pri,
           rel_att, rel_msg, skip, ln_g, ln_b, rte_W, rte_b):
    nt = node_type.astype(jnp.int32)
    nf = jnp.concatenate([node_feature, jnp.zeros((NP - N, N_HID), jnp.float32)])
    nt_p = jnp.concatenate([nt, jnp.zeros((NP - N,), jnp.int32)])
    nt_col = nt_p.reshape(NP, 1)

    pad = EP - E
    src = jnp.concatenate([edge_index[0].astype(jnp.int32), jnp.zeros((pad,), jnp.int32)])
    dst = jnp.concatenate([edge_index[1].astype(jnp.int32), jnp.full((pad,), N, jnp.int32)])
    tim = jnp.concatenate([edge_time.astype(jnp.int32), jnp.zeros((pad,), jnp.int32)])
    rt = jnp.concatenate([edge_type.astype(jnp.int32), jnp.zeros((pad,), jnp.int32)])

    zpage = jnp.zeros((CHUNK, N_HID), jnp.float32)

    x = _adapt(nf, nt_col, W_adapt, b_adapt)

    for l in range(NLAYERS):
        BA = _block_diag(rel_att[l], scale=rel_pri[l] / SQRT_DK)
        BM = _block_diag(rel_msg[l])
        qa = _proj_q(x, nt_col, Wq[l], bq[l], BA)
        c = _proj_c(x, nt_col, Wk[l], Wv[l], BM)
        tbl = _tables(rte_W[l], rte_b[l], Wk[l], bk[l], Wv[l], bv[l], BM)
        agg2, den = _edge_sc(
            qa.reshape(NP * NREL, N_HID),
            c.reshape(NP * NREL, 2 * N_HID),
            tbl.reshape(NTYPES * NREL * MAX_T, 2 * N_HID),
            src, dst, tim, rt, nt_p, zpage)
        alpha = jax.nn.sigmoid(skip[l])
        Wa_eff = Wa[l] * alpha[:, None, None]
        ba_eff = ba[l] * alpha[:, None]
        beta = jnp.broadcast_to((1.0 - alpha)[:, None], (NTYPES, N_HID))
        den3 = den.reshape(2, DEN_ROWS * N_HID)[:, :NP * NHEADS]
        x = _update(agg2, den3.reshape(2, NP, NHEADS), x, nt_col,
                    Wa_eff, ba_eff, beta, ln_g[l], ln_b[l])

    return x[:N]
